# baseline (speedup 1.0000x reference)
import functools

import jax
import jax.numpy as jnp
from jax import lax
from jax.experimental import pallas as pl
from jax.experimental.pallas import tpu as pltpu
from jax.experimental.pallas import tpu_sc as plsc


def _ln(h, g, b):
    m = jnp.mean(h, axis=-1, keepdims=True)
    v = jnp.mean(h * h, axis=-1, keepdims=True) - m * m
    return (h - m) * jax.lax.rsqrt(v + 1e-5) * g + b


def _shift_cat(x):
    z = jnp.zeros((1, x.shape[1]), x.dtype)
    prv = jnp.concatenate([z, x[:-1]], axis=0)
    nxt = jnp.concatenate([x[1:], z], axis=0)
    return jnp.concatenate([prv, x, nxt], axis=1)


def _vp_block(x, w1c, b1, g1, be1, w2c, b2, g2, be2, lw):
    h = jnp.dot(_shift_cat(x), w1c, preferred_element_type=jnp.float32) + b1
    h = _ln(jnp.maximum(h, 0.0), g1, be1)
    h = jnp.dot(_shift_cat(h), w2c, preferred_element_type=jnp.float32) + b2
    h = _ln(jnp.maximum(h, 0.0), g2, be2)
    return jnp.dot(h, lw, preferred_element_type=jnp.float32)


def _tc_body(maxl, dur_ref, x_ref, w1pe_ref, b1pe_ref, *refs):
    dur_p = [r[...] for r in refs[0:9]]
    pit_t = [r[...] for r in refs[9:16]]
    ene_t = [r[...] for r in refs[16:23]]
    ld_ref, pp_ref, ep_ref, csum_ref = refs[23:27]

    s = dur_ref.shape[-1]
    dur = dur_ref[0].astype(jnp.float32)
    i = jax.lax.broadcasted_iota(jnp.int32, (s, s), 0)
    j = jax.lax.broadcasted_iota(jnp.int32, (s, s), 1)
    u = (i <= j).astype(jnp.float32)
    cs = jnp.dot(dur, u, preferred_element_type=jnp.float32)
    prev = cs - dur
    pos = jax.lax.broadcasted_iota(jnp.int32, (maxl, s), 0).astype(jnp.float32)
    e = ((prev <= pos) & (pos < cs)).astype(jnp.float32)
    x_exp = jnp.dot(e, x_ref[0], preferred_element_type=jnp.float32)
    csum_ref[0] = cs.astype(jnp.int32)

    ld_ref[0] = _vp_block(x_ref[0], *dur_p)

    f = pit_t[2].shape[1]
    xce = _shift_cat(x_exp)
    h12 = jnp.maximum(
        jnp.dot(xce, w1pe_ref[...], preferred_element_type=jnp.float32)
        + b1pe_ref[...], 0.0)

    def _vp_tail(h1, g1, be1, w2c, b2, g2, be2, lw):
        h = _ln(h1, g1, be1)
        h = jnp.dot(_shift_cat(h), w2c, preferred_element_type=jnp.float32) + b2
        h = _ln(jnp.maximum(h, 0.0), g2, be2)
        return jnp.dot(h, lw, preferred_element_type=jnp.float32)

    pp_ref[0] = _vp_tail(h12[:, :f], *pit_t)
    ep_ref[0] = _vp_tail(h12[:, f:], *ene_t)


_CHUNK = 32


def _vgather(v, idx):
    return lax.gather(
        v, idx[:, None],
        lax.GatherDimensionNumbers(offset_dims=(), collapsed_slice_dims=(0,),
                                   start_index_map=(0,)),
        (1,), mode=lax.GatherScatterMode.PROMISE_IN_BOUNDS)


_NBUF = 4


def _sc_out_body(s, maxl, h, nb,
                 xpad_ref, dur_ref, pt_ref, et_ref, pb_ref, eb_ref,
                 pemb_ref, eemb_ref, out_ref,
                 dur_v, csum_v, tgtp_v, tgte_v, binsp_v, binse_v,
                 tok_v, idxp_v, idxe_v, *scr):
    bufs = tuple(scr[i * 7:(i + 1) * 7] for i in range(_NBUF))
    b = lax.axis_index("s")
    half = lax.axis_index("c")
    hperw = maxl // 2
    fb = half * hperw
    xbase = b * (s + 1)
    zero_row = xbase + s

    pltpu.sync_copy(dur_ref.at[b], dur_v)
    pltpu.sync_copy(pt_ref.at[pl.ds(b * maxl + fb, hperw)], tgtp_v)
    pltpu.sync_copy(et_ref.at[pl.ds(b * maxl + fb, hperw)], tgte_v)
    pltpu.sync_copy(pb_ref, binsp_v)
    pltpu.sync_copy(eb_ref, binse_v)

    lane = lax.iota(jnp.int32, 16)
    fifteen = jnp.full((16,), 15, jnp.int32)
    carry = jnp.zeros((16,), jnp.int32)
    for r in range(s // 16):
        cs = dur_v[pl.ds(r * 16, 16)]
        for k in (1, 2, 4, 8):
            sh = _vgather(cs, jnp.maximum(lane - k, 0))
            cs = cs + jnp.where(lane >= k, sh, 0)
        cs = cs + carry
        csum_v[pl.ds(r * 16, 16)] = cs
        carry = _vgather(cs, fifteen)

    def tok_body(r, _):
        p = fb + r * 16 + lax.iota(jnp.int32, 16)
        idx = jnp.zeros((16,), jnp.int32)
        for st in (512, 256, 128, 64, 32, 16, 8, 4, 2, 1):
            cand = idx + st
            g = plsc.load_gather(csum_v, [jnp.minimum(cand, s) - 1])
            ok = (cand <= s) & (g <= p)
            idx = jnp.where(ok, cand, idx)
        tok_v[pl.ds(r * 16, 16)] = jnp.where(idx < s, xbase + idx, zero_row)
        return 0

    lax.fori_loop(0, hperw // 16, tok_body, 0)

    def bkt_body(r, _):
        sl = pl.ds(r * 16, 16)
        for tgt_v, bins_v, idx_v in ((tgtp_v, binsp_v, idxp_v),
                                     (tgte_v, binse_v, idxe_v)):
            v = tgt_v[sl]
            idx = jnp.zeros((16,), jnp.int32)
            for st in (256, 128, 64, 32, 16, 8, 4, 2, 1):
                cand = idx + st
                g = plsc.load_gather(bins_v, [jnp.minimum(cand, nb) - 1])
                ok = (cand <= nb) & (g < v)
                idx = jnp.where(ok, cand, idx)
            idx_v[sl] = idx
        return 0

    lax.fori_loop(0, hperw // 16, bkt_body, 0)

    obase = b * maxl + fb
    nch = hperw // _CHUNK
    nbuf = len(bufs)
    pend_g = [None] * nbuf
    pend_w = [None] * nbuf

    def fire(ch, xr, pr, er, s1, s2, s3):
        rsl = pl.ds(ch * _CHUNK, _CHUNK)
        return (pltpu.async_copy(xpad_ref.at[tok_v.at[rsl]], xr, s1),
                pltpu.async_copy(pemb_ref.at[idxp_v.at[rsl]], pr, s2),
                pltpu.async_copy(eemb_ref.at[idxe_v.at[rsl]], er, s3))

    for ch in range(nch + nbuf - 1):
        i = ch % nbuf
        if ch < nch:
            if pend_w[i] is not None:
                pend_w[i].wait()
                pend_w[i] = None
            xr, pr, er, s1, s2, s3, ws = bufs[i]
            pend_g[i] = fire(ch, xr, pr, er, s1, s2, s3)
        pc = ch - nbuf + 1
        if pc >= 0:
            j = pc % nbuf
            for cpy in pend_g[j]:
                cpy.wait()
            xr, pr, er, s1, s2, s3, ws = bufs[j]

            def add_body(r, _, xr=xr, pr=pr, er=er):
                for jj in range(h // 16):
                    csl = pl.ds(jj * 16, 16)
                    xr[r, csl] = xr[r, csl] + pr[r, csl] + er[r, csl]
                return 0

            lax.fori_loop(0, _CHUNK, add_body, 0)
            pend_w[j] = pltpu.async_copy(
                xr, out_ref.at[pl.ds(obase + pc * _CHUNK, _CHUNK)], ws)
    for i in range(nbuf):
        if pend_w[i] is not None:
            pend_w[i].wait()


def _full(shape, dtype=jnp.float32):
    return pl.BlockSpec(shape, lambda b: (0,) * len(shape))


def _batched(shape):
    return pl.BlockSpec((1,) + shape, lambda b: (b,) + (0,) * len(shape))


def _vp_flat(p, h):
    f = p['w1'].shape[2]
    return (
        p['w1'].reshape(3 * h, f),
        p['b1'].reshape(1, f),
        p['g1'].reshape(1, f),
        p['be1'].reshape(1, f),
        p['w2'].reshape(3 * f, f),
        p['b2'].reshape(1, f),
        p['g2'].reshape(1, f),
        p['be2'].reshape(1, f),
        p['lw'],
    )


def kernel(x, src_mask, mel_mask, duration_target, pitch_target,
           energy_target, max_len, params):
    b, s, h = x.shape
    maxl = mel_mask.shape[1]
    nb = params['pitch_emb'].shape[0]

    cp = pltpu.CompilerParams(dimension_semantics=("parallel",))

    dur_p = _vp_flat(params['dur'], h)
    pit_p = _vp_flat(params['pitch'], h)
    ene_p = _vp_flat(params['energy'], h)
    w1pe = jnp.concatenate([pit_p[0], ene_p[0]], axis=1)
    b1pe = jnp.concatenate([pit_p[1], ene_p[1]], axis=1)
    pit_t = pit_p[2:]
    ene_t = ene_p[2:]
    param_specs = [_full(a.shape) for a in dur_p + pit_t + ene_t]

    log_dur, pitch_pred, energy_pred, csum = pl.pallas_call(
        functools.partial(_tc_body, maxl),
        grid=(b,),
        in_specs=[_batched((1, s)), _batched((s, h)),
                  _full(w1pe.shape), _full(b1pe.shape)] + param_specs,
        out_specs=[_batched((s, 1)), _batched((maxl, 1)), _batched((maxl, 1)),
                   _batched((1, s))],
        out_shape=[jax.ShapeDtypeStruct((b, s, 1), jnp.float32),
                   jax.ShapeDtypeStruct((b, maxl, 1), jnp.float32),
                   jax.ShapeDtypeStruct((b, maxl, 1), jnp.float32),
                   jax.ShapeDtypeStruct((b, 1, s), jnp.int32)],
        compiler_params=cp,
    )(duration_target.reshape(b, 1, s), x, w1pe, b1pe,
      *dur_p, *pit_t, *ene_t)

    inf_pad = jnp.full((1,), jnp.inf, jnp.float32)
    pbins1 = jnp.concatenate([params['pitch_bins'], inf_pad])
    ebins1 = jnp.concatenate([params['energy_bins'], inf_pad])
    xpad = jnp.concatenate(
        [x, jnp.zeros((b, 1, h), jnp.float32)], axis=1).reshape(b * (s + 1), h)

    mesh = plsc.VectorSubcoreMesh(core_axis_name="c", subcore_axis_name="s")
    sc_out = pl.kernel(
        functools.partial(_sc_out_body, s, maxl, h, nb),
        mesh=mesh,
        compiler_params=pltpu.CompilerParams(needs_layout_passes=False),
        out_type=jax.ShapeDtypeStruct((b * maxl, h), jnp.float32),
        scratch_types=[
            pltpu.VMEM((s,), jnp.int32),
            pltpu.VMEM((s,), jnp.int32),
            pltpu.VMEM((maxl // 2,), jnp.float32),
            pltpu.VMEM((maxl // 2,), jnp.float32),
            pltpu.VMEM((nb,), jnp.float32),
            pltpu.VMEM((nb,), jnp.float32),
            pltpu.VMEM((maxl // 2,), jnp.int32),
            pltpu.VMEM((maxl // 2,), jnp.int32),
            pltpu.VMEM((maxl // 2,), jnp.int32),
        ] + [
            st
            for _ in range(_NBUF)
            for st in (pltpu.VMEM((_CHUNK, h), jnp.float32),
                       pltpu.VMEM((_CHUNK, h), jnp.float32),
                       pltpu.VMEM((_CHUNK, h), jnp.float32),
                       pltpu.SemaphoreType.DMA,
                       pltpu.SemaphoreType.DMA,
                       pltpu.SemaphoreType.DMA,
                       pltpu.SemaphoreType.DMA)
        ],
    )
    out = sc_out(xpad, duration_target, pitch_target.reshape(b * maxl),
                 energy_target.reshape(b * maxl), pbins1, ebins1,
                 params['pitch_emb'], params['energy_emb'])
    out = out.reshape(b, maxl, h)

    mel_len = csum[:, 0, -1]
    log_dur = log_dur[:, :, 0] + params['dur']['lb'][0]
    pitch_pred = pitch_pred[:, :, 0] + params['pitch']['lb'][0]
    energy_pred = energy_pred[:, :, 0] + params['energy']['lb'][0]
    log_dur = jnp.where(src_mask, 0.0, log_dur)
    pitch_pred = jnp.where(mel_mask, 0.0, pitch_pred)
    energy_pred = jnp.where(mel_mask, 0.0, energy_pred)

    return (out, log_dur, pitch_pred, energy_pred, mel_len, mel_mask)


# speedup vs baseline: 1.0734x; 1.0734x over previous
import functools

import jax
import jax.numpy as jnp
from jax import lax
from jax.experimental import pallas as pl
from jax.experimental.pallas import tpu as pltpu
from jax.experimental.pallas import tpu_sc as plsc


def _ln(h, g, b):
    m = jnp.mean(h, axis=-1, keepdims=True)
    v = jnp.mean(h * h, axis=-1, keepdims=True) - m * m
    return (h - m) * jax.lax.rsqrt(v + 1e-5) * g + b


def _shift_cat(x):
    z = jnp.zeros((1, x.shape[1]), x.dtype)
    prv = jnp.concatenate([z, x[:-1]], axis=0)
    nxt = jnp.concatenate([x[1:], z], axis=0)
    return jnp.concatenate([prv, x, nxt], axis=1)


def _vp_block(x, w1c, b1, g1, be1, w2c, b2, g2, be2, lw):
    h = jnp.dot(_shift_cat(x), w1c, preferred_element_type=jnp.float32) + b1
    h = _ln(jnp.maximum(h, 0.0), g1, be1)
    h = jnp.dot(_shift_cat(h), w2c, preferred_element_type=jnp.float32) + b2
    h = _ln(jnp.maximum(h, 0.0), g2, be2)
    return jnp.dot(h, lw, preferred_element_type=jnp.float32)


def _tc_body(maxl, dur_ref, x_ref, w1pe_ref, b1pe_ref, *refs):
    dur_p = [r[...] for r in refs[0:9]]
    pit_t = [r[...] for r in refs[9:16]]
    ene_t = [r[...] for r in refs[16:23]]
    ld_ref, pp_ref, ep_ref, csum_ref = refs[23:27]

    s = dur_ref.shape[-1]
    dur = dur_ref[0].astype(jnp.float32)
    i = jax.lax.broadcasted_iota(jnp.int32, (s, s), 0)
    j = jax.lax.broadcasted_iota(jnp.int32, (s, s), 1)
    u = (i <= j).astype(jnp.float32)
    cs = jnp.dot(dur, u, preferred_element_type=jnp.float32)
    prev = cs - dur
    pos = jax.lax.broadcasted_iota(jnp.int32, (maxl, s), 0).astype(jnp.float32)
    e = ((prev <= pos) & (pos < cs)).astype(jnp.float32)
    x_exp = jnp.dot(e, x_ref[0], preferred_element_type=jnp.float32)
    csum_ref[0] = cs.astype(jnp.int32)

    ld_ref[0] = _vp_block(x_ref[0], *dur_p)

    f = pit_t[2].shape[1]
    xce = _shift_cat(x_exp)
    h12 = jnp.maximum(
        jnp.dot(xce, w1pe_ref[...], preferred_element_type=jnp.float32)
        + b1pe_ref[...], 0.0)

    def _vp_tail(h1, g1, be1, w2c, b2, g2, be2, lw):
        h = _ln(h1, g1, be1)
        h = jnp.dot(_shift_cat(h), w2c, preferred_element_type=jnp.float32) + b2
        h = _ln(jnp.maximum(h, 0.0), g2, be2)
        return jnp.dot(h, lw, preferred_element_type=jnp.float32)

    pp_ref[0] = _vp_tail(h12[:, :f], *pit_t)
    ep_ref[0] = _vp_tail(h12[:, f:], *ene_t)


_CHUNK = 64


def _vgather(v, idx):
    return lax.gather(
        v, idx[:, None],
        lax.GatherDimensionNumbers(offset_dims=(), collapsed_slice_dims=(0,),
                                   start_index_map=(0,)),
        (1,), mode=lax.GatherScatterMode.PROMISE_IN_BOUNDS)


_NBUF = 2


def _sc_out_body(s, maxl, h, nb,
                 xpad_ref, dur_ref, pt_ref, et_ref, pb_ref, eb_ref,
                 pemb_ref, eemb_ref, out_ref,
                 dur_v, csum_v, tgtp_v, tgte_v, binsp_v, binse_v,
                 tok_v, idxp_v, idxe_v, *scr):
    bufs = tuple(scr[i * 7:(i + 1) * 7] for i in range(_NBUF))
    b = lax.axis_index("s")
    half = lax.axis_index("c")
    hperw = maxl // 2
    fb = half * hperw
    xbase = b * (s + 1)
    zero_row = xbase + s

    pltpu.sync_copy(dur_ref.at[b], dur_v)
    pltpu.sync_copy(pt_ref.at[pl.ds(b * maxl + fb, hperw)], tgtp_v)
    pltpu.sync_copy(et_ref.at[pl.ds(b * maxl + fb, hperw)], tgte_v)
    pltpu.sync_copy(pb_ref, binsp_v)
    pltpu.sync_copy(eb_ref, binse_v)

    lane = lax.iota(jnp.int32, 16)
    fifteen = jnp.full((16,), 15, jnp.int32)
    carry = jnp.zeros((16,), jnp.int32)
    for r in range(s // 16):
        cs = dur_v[pl.ds(r * 16, 16)]
        for k in (1, 2, 4, 8):
            sh = _vgather(cs, jnp.maximum(lane - k, 0))
            cs = cs + jnp.where(lane >= k, sh, 0)
        cs = cs + carry
        csum_v[pl.ds(r * 16, 16)] = cs
        carry = _vgather(cs, fifteen)

    def tok_body(r, _):
        p = fb + r * 16 + lax.iota(jnp.int32, 16)
        idx = jnp.zeros((16,), jnp.int32)
        for st in (512, 256, 128, 64, 32, 16, 8, 4, 2, 1):
            cand = idx + st
            g = plsc.load_gather(csum_v, [jnp.minimum(cand, s) - 1])
            ok = (cand <= s) & (g <= p)
            idx = jnp.where(ok, cand, idx)
        tok_v[pl.ds(r * 16, 16)] = jnp.where(idx < s, xbase + idx, zero_row)
        return 0

    def bkt_body(r, _):
        sl = pl.ds(r * 16, 16)
        for tgt_v, bins_v, idx_v in ((tgtp_v, binsp_v, idxp_v),
                                     (tgte_v, binse_v, idxe_v)):
            v = tgt_v[sl]
            idx = jnp.zeros((16,), jnp.int32)
            for st in (256, 128, 64, 32, 16, 8, 4, 2, 1):
                cand = idx + st
                g = plsc.load_gather(bins_v, [jnp.minimum(cand, nb) - 1])
                ok = (cand <= nb) & (g < v)
                idx = jnp.where(ok, cand, idx)
            idx_v[sl] = idx
        return 0

    vper = _CHUNK // 16

    def idx_chunk(ch):
        lax.fori_loop(ch * vper, (ch + 1) * vper, tok_body, 0)
        lax.fori_loop(ch * vper, (ch + 1) * vper, bkt_body, 0)

    obase = b * maxl + fb
    nch = hperw // _CHUNK
    nbuf = len(bufs)
    pend_g = [None] * nbuf
    pend_w = [None] * nbuf

    def fire(ch, xr, pr, er, s1, s2, s3):
        rsl = pl.ds(ch * _CHUNK, _CHUNK)
        return (pltpu.async_copy(xpad_ref.at[tok_v.at[rsl]], xr, s1),
                pltpu.async_copy(pemb_ref.at[idxp_v.at[rsl]], pr, s2),
                pltpu.async_copy(eemb_ref.at[idxe_v.at[rsl]], er, s3))

    for ch in range(nch + nbuf - 1):
        i = ch % nbuf
        if ch < nch:
            if pend_w[i] is not None:
                pend_w[i].wait()
                pend_w[i] = None
            idx_chunk(ch)
            xr, pr, er, s1, s2, s3, ws = bufs[i]
            pend_g[i] = fire(ch, xr, pr, er, s1, s2, s3)
        pc = ch - nbuf + 1
        if pc >= 0:
            j = pc % nbuf
            for cpy in pend_g[j]:
                cpy.wait()
            xr, pr, er, s1, s2, s3, ws = bufs[j]

            def add_body(r, _, xr=xr, pr=pr, er=er):
                for jj in range(h // 16):
                    csl = pl.ds(jj * 16, 16)
                    xr[r, csl] = xr[r, csl] + pr[r, csl] + er[r, csl]
                return 0

            lax.fori_loop(0, _CHUNK, add_body, 0)
            pend_w[j] = pltpu.async_copy(
                xr, out_ref.at[pl.ds(obase + pc * _CHUNK, _CHUNK)], ws)
    for i in range(nbuf):
        if pend_w[i] is not None:
            pend_w[i].wait()


def _full(shape, dtype=jnp.float32):
    return pl.BlockSpec(shape, lambda b: (0,) * len(shape))


def _batched(shape):
    return pl.BlockSpec((1,) + shape, lambda b: (b,) + (0,) * len(shape))


def _vp_flat(p, h):
    f = p['w1'].shape[2]
    return (
        p['w1'].reshape(3 * h, f),
        p['b1'].reshape(1, f),
        p['g1'].reshape(1, f),
        p['be1'].reshape(1, f),
        p['w2'].reshape(3 * f, f),
        p['b2'].reshape(1, f),
        p['g2'].reshape(1, f),
        p['be2'].reshape(1, f),
        p['lw'],
    )


def kernel(x, src_mask, mel_mask, duration_target, pitch_target,
           energy_target, max_len, params):
    b, s, h = x.shape
    maxl = mel_mask.shape[1]
    nb = params['pitch_emb'].shape[0]

    cp = pltpu.CompilerParams(dimension_semantics=("parallel",))

    dur_p = _vp_flat(params['dur'], h)
    pit_p = _vp_flat(params['pitch'], h)
    ene_p = _vp_flat(params['energy'], h)
    w1pe = jnp.concatenate([pit_p[0], ene_p[0]], axis=1)
    b1pe = jnp.concatenate([pit_p[1], ene_p[1]], axis=1)
    pit_t = pit_p[2:]
    ene_t = ene_p[2:]
    param_specs = [_full(a.shape) for a in dur_p + pit_t + ene_t]

    log_dur, pitch_pred, energy_pred, csum = pl.pallas_call(
        functools.partial(_tc_body, maxl),
        grid=(b,),
        in_specs=[_batched((1, s)), _batched((s, h)),
                  _full(w1pe.shape), _full(b1pe.shape)] + param_specs,
        out_specs=[_batched((s, 1)), _batched((maxl, 1)), _batched((maxl, 1)),
                   _batched((1, s))],
        out_shape=[jax.ShapeDtypeStruct((b, s, 1), jnp.float32),
                   jax.ShapeDtypeStruct((b, maxl, 1), jnp.float32),
                   jax.ShapeDtypeStruct((b, maxl, 1), jnp.float32),
                   jax.ShapeDtypeStruct((b, 1, s), jnp.int32)],
        compiler_params=cp,
    )(duration_target.reshape(b, 1, s), x, w1pe, b1pe,
      *dur_p, *pit_t, *ene_t)

    inf_pad = jnp.full((1,), jnp.inf, jnp.float32)
    pbins1 = jnp.concatenate([params['pitch_bins'], inf_pad])
    ebins1 = jnp.concatenate([params['energy_bins'], inf_pad])
    xpad = jnp.concatenate(
        [x, jnp.zeros((b, 1, h), jnp.float32)], axis=1).reshape(b * (s + 1), h)

    mesh = plsc.VectorSubcoreMesh(core_axis_name="c", subcore_axis_name="s")
    sc_out = pl.kernel(
        functools.partial(_sc_out_body, s, maxl, h, nb),
        mesh=mesh,
        compiler_params=pltpu.CompilerParams(needs_layout_passes=False),
        out_type=jax.ShapeDtypeStruct((b * maxl, h), jnp.float32),
        scratch_types=[
            pltpu.VMEM((s,), jnp.int32),
            pltpu.VMEM((s,), jnp.int32),
            pltpu.VMEM((maxl // 2,), jnp.float32),
            pltpu.VMEM((maxl // 2,), jnp.float32),
            pltpu.VMEM((nb,), jnp.float32),
            pltpu.VMEM((nb,), jnp.float32),
            pltpu.VMEM((maxl // 2,), jnp.int32),
            pltpu.VMEM((maxl // 2,), jnp.int32),
            pltpu.VMEM((maxl // 2,), jnp.int32),
        ] + [
            st
            for _ in range(_NBUF)
            for st in (pltpu.VMEM((_CHUNK, h), jnp.float32),
                       pltpu.VMEM((_CHUNK, h), jnp.float32),
                       pltpu.VMEM((_CHUNK, h), jnp.float32),
                       pltpu.SemaphoreType.DMA,
                       pltpu.SemaphoreType.DMA,
                       pltpu.SemaphoreType.DMA,
                       pltpu.SemaphoreType.DMA)
        ],
    )
    out = sc_out(xpad, duration_target, pitch_target.reshape(b * maxl),
                 energy_target.reshape(b * maxl), pbins1, ebins1,
                 params['pitch_emb'], params['energy_emb'])
    out = out.reshape(b, maxl, h)

    mel_len = csum[:, 0, -1]
    log_dur = log_dur[:, :, 0] + params['dur']['lb'][0]
    pitch_pred = pitch_pred[:, :, 0] + params['pitch']['lb'][0]
    energy_pred = energy_pred[:, :, 0] + params['energy']['lb'][0]
    log_dur = jnp.where(src_mask, 0.0, log_dur)
    pitch_pred = jnp.where(mel_mask, 0.0, pitch_pred)
    energy_pred = jnp.where(mel_mask, 0.0, energy_pred)

    return (out, log_dur, pitch_pred, energy_pred, mel_len, mel_mask)
